# baseline (speedup 1.0000x reference)
import functools

import jax
import jax.numpy as jnp
from jax import lax
from jax.experimental import pallas as pl
from jax.experimental.pallas import tpu as pltpu
from jax.experimental.pallas import tpu_sc as plsc

_NC = 2
_NS = 16
_NW = _NC * _NS
_CHUNK_ROWS = 8
_NBUF = 4
_LANES = 16
_DBLOCK = 512


def _make_sc_kernel(rows, D):
    rows_per_w = rows // _NW
    n_chunks = rows_per_w // _CHUNK_ROWS
    assert n_chunks % _NBUF == 0
    chunk_elems = _CHUNK_ROWS * D
    mesh = plsc.VectorSubcoreMesh(core_axis_name="c", subcore_axis_name="s")

    @functools.partial(
        pl.kernel,
        mesh=mesh,
        out_type=jax.ShapeDtypeStruct((rows * D,), jnp.float32),
        scratch_types=[
            pltpu.VMEM((16,), jnp.int32),
            pltpu.VMEM((D,), jnp.float32),
            pltpu.VMEM((_NBUF, chunk_elems), jnp.float32),
            pltpu.SemaphoreType.DMA((_NBUF,)),
            pltpu.SemaphoreType.DMA((_NBUF,)),
        ],
    )
    def sc_kernel(idx_hbm, x_hbm, emb_hbm, out_hbm, idx_v, emb_v, buf_v,
                  sem_in, sem_out):
        wid = lax.axis_index("s") * _NC + lax.axis_index("c")
        base = wid * rows_per_w * D
        pltpu.sync_copy(idx_hbm, idx_v)
        row_idx = idx_v[pl.ds(0, _LANES)][0]
        pltpu.sync_copy(emb_hbm.at[pl.ds(row_idx * D, D)], emb_v)

        def in_copy(g, b):
            return pltpu.make_async_copy(
                x_hbm.at[pl.ds(base + g * chunk_elems, chunk_elems)],
                buf_v.at[b],
                sem_in.at[b],
            )

        def out_copy(g, b):
            return pltpu.make_async_copy(
                buf_v.at[b],
                out_hbm.at[pl.ds(base + g * chunk_elems, chunk_elems)],
                sem_out.at[b],
            )

        in_copy(0, 0).start()
        in_copy(1, 1).start()

        def quad_body(g0):
            for k in range(_NBUF):
                g = g0 + k
                b_next = (k + 2) % _NBUF
                pl.when(g >= 2)(lambda g=g, b=b_next: out_copy(g - 2, b).wait())
                pl.when(g + 2 < n_chunks)(
                    lambda g=g, b=b_next: in_copy(g + 2, b).start()
                )
                in_copy(g, k).wait()
                bufk = buf_v.at[k]
                for db in range(D // _DBLOCK):
                    ev = [
                        emb_v[pl.ds(db * _DBLOCK + j * _LANES, _LANES)]
                        for j in range(_DBLOCK // _LANES)
                    ]

                    def row_body(r, db=db, ev=ev):
                        rb = r * D + db * _DBLOCK
                        for j in range(_DBLOCK // _LANES):
                            sl = pl.ds(rb + j * _LANES, _LANES)
                            bufk[sl] = bufk[sl] + ev[j]

                    pl.loop(0, _CHUNK_ROWS)(row_body)
                out_copy(g, k).start()

        pl.loop(0, n_chunks, step=_NBUF)(quad_body)
        out_copy(n_chunks - 2, (n_chunks - 2) % _NBUF).wait()
        out_copy(n_chunks - 1, (n_chunks - 1) % _NBUF).wait()

    return sc_kernel


def kernel(x, layer_idx, step_embeddings):
    B, S, D = x.shape
    rows = B * S
    n_table = step_embeddings.shape[0]
    x_flat = x.reshape(rows * D)
    emb_flat = step_embeddings.reshape(n_table * D)
    idx = jnp.asarray(layer_idx, dtype=jnp.int32).reshape(1)
    idx16 = jnp.pad(idx, (0, 15))
    sc = _make_sc_kernel(rows, D)
    out = sc(idx16, x_flat, emb_flat)
    return out.reshape(B, S, D)


# speedup vs baseline: 4.6042x; 4.6042x over previous
import jax
import jax.numpy as jnp
from jax.experimental import pallas as pl
from jax.experimental.pallas import tpu as pltpu

_CHUNK_ROWS = 1024
_NBUF = 4


def _add_body(idx_ref, x_hbm, emb_ref, o_hbm, buf, sin, sout):
    n_chunks = x_hbm.shape[0] // _CHUNK_ROWS
    row = emb_ref[idx_ref[0]]

    def in_c(g):
        return pltpu.make_async_copy(
            x_hbm.at[pl.ds(g * _CHUNK_ROWS, _CHUNK_ROWS)],
            buf.at[g % _NBUF],
            sin.at[g % _NBUF],
        )

    def out_c(g):
        return pltpu.make_async_copy(
            buf.at[g % _NBUF],
            o_hbm.at[pl.ds(g * _CHUNK_ROWS, _CHUNK_ROWS)],
            sout.at[g % _NBUF],
        )

    in_c(0).start()
    if n_chunks > 1:
        in_c(1).start()
    for g in range(n_chunks):
        if g + 2 < n_chunks:
            if g >= 2:
                out_c(g - 2).wait()
            in_c(g + 2).start()
        in_c(g).wait()
        buf[g % _NBUF] = buf[g % _NBUF] + row
        out_c(g).start()
    for g in range(max(0, n_chunks - _NBUF), n_chunks):
        out_c(g).wait()


def kernel(x, layer_idx, step_embeddings):
    B, S, D = x.shape
    rows = B * S
    x2 = x.reshape(rows, D)
    n_table = step_embeddings.shape[0]
    idx = jnp.asarray(layer_idx, dtype=jnp.int32).reshape(1)
    out = pl.pallas_call(
        _add_body,
        in_specs=[
            pl.BlockSpec(memory_space=pltpu.SMEM),
            pl.BlockSpec(memory_space=pl.ANY),
            pl.BlockSpec(memory_space=pltpu.VMEM),
        ],
        out_specs=pl.BlockSpec(memory_space=pl.ANY),
        out_shape=jax.ShapeDtypeStruct((rows, D), x.dtype),
        scratch_shapes=[
            pltpu.VMEM((_NBUF, _CHUNK_ROWS, D), jnp.float32),
            pltpu.SemaphoreType.DMA((_NBUF,)),
            pltpu.SemaphoreType.DMA((_NBUF,)),
        ],
    )(idx, x2, step_embeddings)
    return out.reshape(B, S, D)
